# baseline (speedup 1.0000x reference)
import functools

import jax
import jax.numpy as jnp
from jax import lax
from jax.experimental import pallas as pl
from jax.experimental.pallas import tpu as pltpu
from jax.experimental.pallas import tpu_sc as plsc

VOCAB = 1000000
DIM = 32
PACKED = DIM // 2
BATCH = 16384

NUM_CORES = 2
NUM_SUBCORES = 16
LANES = 16
NW = NUM_CORES * NUM_SUBCORES
BPW = BATCH // NW
GROUPS = BPW // LANES

_DNUMS = jax.lax.GatherDimensionNumbers(
    offset_dims=(), collapsed_slice_dims=(0,), start_index_map=(0,))


def _permute(v, idx):
    return jax.lax.gather(
        v, idx.reshape(LANES, 1), _DNUMS, slice_sizes=(1,),
        mode=jax.lax.GatherScatterMode.PROMISE_IN_BOUNDS)


def _unpack(x):
    even = lax.bitcast_convert_type(x << 16, jnp.float32)
    odd = lax.bitcast_convert_type(x & jnp.int32(-65536), jnp.float32)
    return even, odd


def _sc_kernel_body(word_hbm, ctx_hbm, ht_hbm, ct_hbm, out_hbm,
                    wi_v, ci_v, wr_v, cr_v, o_v, sem_w, sem_c):
    wid = lax.axis_index("s") * NUM_CORES + lax.axis_index("c")
    base = wid * BPW

    pltpu.sync_copy(word_hbm.at[pl.ds(base, BPW)], wi_v)
    pltpu.sync_copy(ctx_hbm.at[pl.ds(base, BPW)], ci_v)
    cp_w = pltpu.async_copy(ht_hbm.at[wi_v], wr_v, sem_w)
    cp_c = pltpu.async_copy(ct_hbm.at[ci_v], cr_v, sem_c)
    cp_w.wait()
    cp_c.wait()

    def group_body(g, _):
        row0 = g * LANES
        acc = jnp.zeros((LANES,), jnp.float32)
        for i in range(LANES):
            r = row0 + i
            wa, wb = _unpack(wr_v[r, pl.ds(0, PACKED)])
            ca, cb = _unpack(cr_v[r, pl.ds(0, PACKED)])
            p = wa * ca + wb * cb
            for sh in (8, 4, 2, 1):
                p = p + _permute(p, jnp.bitwise_xor(lax.iota(jnp.int32, LANES), sh))
            acc = jnp.where(lax.iota(jnp.int32, LANES) == i, p, acc)
        o_v[pl.ds(row0, LANES)] = 1.0 / (1.0 + jnp.exp(-acc))
        return 0

    lax.fori_loop(0, GROUPS, group_body, 0)
    pltpu.sync_copy(o_v, out_hbm.at[pl.ds(base, BPW)])


_sc_call = functools.partial(
    pl.kernel,
    out_type=jax.ShapeDtypeStruct((BATCH,), jnp.float32),
    mesh=plsc.VectorSubcoreMesh(
        core_axis_name="c", subcore_axis_name="s",
        num_cores=NUM_CORES, num_subcores=NUM_SUBCORES),
    compiler_params=pltpu.CompilerParams(use_tc_tiling_on_sc=False),
    scratch_types=[
        pltpu.VMEM((BPW,), jnp.int32),
        pltpu.VMEM((BPW,), jnp.int32),
        pltpu.VMEM((BPW, PACKED), jnp.int32),
        pltpu.VMEM((BPW, PACKED), jnp.int32),
        pltpu.VMEM((BPW,), jnp.float32),
        pltpu.SemaphoreType.DMA,
        pltpu.SemaphoreType.DMA,
    ],
)(_sc_kernel_body)


def _pack(table):
    b = table.astype(jnp.bfloat16).reshape(VOCAB, PACKED, 2)
    return lax.bitcast_convert_type(b, jnp.int32)


def kernel(word, context, hidden_table, context_table):
    out = _sc_call(word.astype(jnp.int32), context.astype(jnp.int32),
                   _pack(hidden_table), _pack(context_table))
    return out.reshape(BATCH, 1)


# speedup vs baseline: 1.3158x; 1.3158x over previous
import functools

import jax
import jax.numpy as jnp
from jax import lax
from jax.experimental import pallas as pl
from jax.experimental.pallas import tpu as pltpu
from jax.experimental.pallas import tpu_sc as plsc

VOCAB = 1000000
HALF = VOCAB // 2
DIM = 32
BATCH = 16384

NUM_CORES = 2
NUM_SUBCORES = 16
LANES = 16
NW = NUM_CORES * NUM_SUBCORES
BPW = BATCH // NW
GROUPS = BPW // LANES

_DNUMS = jax.lax.GatherDimensionNumbers(
    offset_dims=(), collapsed_slice_dims=(0,), start_index_map=(0,))


def _permute(v, idx):
    return jax.lax.gather(
        v, idx.reshape(LANES, 1), _DNUMS, slice_sizes=(1,),
        mode=jax.lax.GatherScatterMode.PROMISE_IN_BOUNDS)


def _sc_kernel_body(word_hbm, ctx_hbm, h0_hbm, h1_hbm, c0_hbm, c1_hbm,
                    out_hbm, wi_v, ci_v, wq0_v, wq1_v, cq0_v, cq1_v,
                    w0_v, w1_v, x0_v, x1_v, o_v,
                    sem_w0, sem_w1, sem_c0, sem_c1):
    wid = lax.axis_index("s") * NUM_CORES + lax.axis_index("c")
    base = wid * BPW

    pltpu.sync_copy(word_hbm.at[pl.ds(base, BPW)], wi_v)
    pltpu.sync_copy(ctx_hbm.at[pl.ds(base, BPW)], ci_v)

    def prep_body(g, _):
        sl = pl.ds(g * LANES, LANES)
        w = wi_v[sl]
        c = ci_v[sl]
        zero = jnp.zeros((LANES,), jnp.int32)
        wq0_v[sl] = jnp.where(w < HALF, w, zero)
        wq1_v[sl] = jnp.where(w < HALF, zero, w - HALF)
        cq0_v[sl] = jnp.where(c < HALF, c, zero)
        cq1_v[sl] = jnp.where(c < HALF, zero, c - HALF)
        return 0

    lax.fori_loop(0, GROUPS, prep_body, 0)

    cp_w0 = pltpu.async_copy(h0_hbm.at[wq0_v], w0_v, sem_w0)
    cp_w1 = pltpu.async_copy(h1_hbm.at[wq1_v], w1_v, sem_w1)
    cp_c0 = pltpu.async_copy(c0_hbm.at[cq0_v], x0_v, sem_c0)
    cp_c1 = pltpu.async_copy(c1_hbm.at[cq1_v], x1_v, sem_c1)
    cp_w0.wait()
    cp_w1.wait()
    cp_c0.wait()
    cp_c1.wait()

    def group_body(g, _):
        row0 = g * LANES
        wi_g = wi_v[pl.ds(row0, LANES)]
        ci_g = ci_v[pl.ds(row0, LANES)]
        acc = jnp.zeros((LANES,), jnp.float32)
        for i in range(LANES):
            r = row0 + i
            pw = wi_g[i] < HALF
            pc = ci_g[i] < HALF
            wa = jnp.where(pw, w0_v[r, pl.ds(0, LANES)],
                           w1_v[r, pl.ds(0, LANES)])
            wb = jnp.where(pw, w0_v[r, pl.ds(LANES, LANES)],
                           w1_v[r, pl.ds(LANES, LANES)])
            ca = jnp.where(pc, x0_v[r, pl.ds(0, LANES)],
                           x1_v[r, pl.ds(0, LANES)])
            cb = jnp.where(pc, x0_v[r, pl.ds(LANES, LANES)],
                           x1_v[r, pl.ds(LANES, LANES)])
            p = wa * ca + wb * cb
            for sh in (8, 4, 2, 1):
                p = p + _permute(p, jnp.bitwise_xor(lax.iota(jnp.int32, LANES), sh))
            acc = jnp.where(lax.iota(jnp.int32, LANES) == i, p, acc)
        o_v[pl.ds(row0, LANES)] = 1.0 / (1.0 + jnp.exp(-acc))
        return 0

    lax.fori_loop(0, GROUPS, group_body, 0)
    pltpu.sync_copy(o_v, out_hbm.at[pl.ds(base, BPW)])


_sc_call = functools.partial(
    pl.kernel,
    out_type=jax.ShapeDtypeStruct((BATCH,), jnp.float32),
    mesh=plsc.VectorSubcoreMesh(
        core_axis_name="c", subcore_axis_name="s",
        num_cores=NUM_CORES, num_subcores=NUM_SUBCORES),
    compiler_params=pltpu.CompilerParams(use_tc_tiling_on_sc=False),
    scratch_types=[
        pltpu.VMEM((BPW,), jnp.int32),
        pltpu.VMEM((BPW,), jnp.int32),
        pltpu.VMEM((BPW,), jnp.int32),
        pltpu.VMEM((BPW,), jnp.int32),
        pltpu.VMEM((BPW,), jnp.int32),
        pltpu.VMEM((BPW,), jnp.int32),
        pltpu.VMEM((BPW, DIM), jnp.float32),
        pltpu.VMEM((BPW, DIM), jnp.float32),
        pltpu.VMEM((BPW, DIM), jnp.float32),
        pltpu.VMEM((BPW, DIM), jnp.float32),
        pltpu.VMEM((BPW,), jnp.float32),
        pltpu.SemaphoreType.DMA,
        pltpu.SemaphoreType.DMA,
        pltpu.SemaphoreType.DMA,
        pltpu.SemaphoreType.DMA,
    ],
)(_sc_kernel_body)


def kernel(word, context, hidden_table, context_table):
    out = _sc_call(word.astype(jnp.int32), context.astype(jnp.int32),
                   hidden_table[:HALF], hidden_table[HALF:],
                   context_table[:HALF], context_table[HALF:])
    return out.reshape(BATCH, 1)


# speedup vs baseline: 2.1631x; 1.6440x over previous
import functools

import jax
import jax.numpy as jnp
from jax import lax
from jax.experimental import pallas as pl
from jax.experimental.pallas import tpu as pltpu
from jax.experimental.pallas import tpu_sc as plsc

VOCAB = 1000000
DIM = 32
BATCH = 16384

NUM_CORES = 2
NUM_SUBCORES = 16
LANES = 16
NW = NUM_CORES * NUM_SUBCORES
BPW = BATCH // NW
CHUNK = 256
NCHUNKS = BPW // CHUNK
CGROUPS = CHUNK // LANES

WIDE = 128
RPW = WIDE // DIM
NWIDE = VOCAB // RPW

_DNUMS = jax.lax.GatherDimensionNumbers(
    offset_dims=(), collapsed_slice_dims=(0,), start_index_map=(0,))


def _permute(v, idx):
    return jax.lax.gather(
        v, idx.reshape(LANES, 1), _DNUMS, slice_sizes=(1,),
        mode=jax.lax.GatherScatterMode.PROMISE_IN_BOUNDS)


def _select_subrow(rows_v, r, sub):
    lo = rows_v[r, pl.ds(0, LANES)]
    hi = rows_v[r, pl.ds(LANES, LANES)]
    for s in range(1, RPW):
        pred = sub == s
        lo = jnp.where(pred, rows_v[r, pl.ds(s * DIM, LANES)], lo)
        hi = jnp.where(pred, rows_v[r, pl.ds(s * DIM + LANES, LANES)], hi)
    return lo, hi


def _sc_kernel_body(word_hbm, ctx_hbm, ht_hbm, ct_hbm, out_hbm,
                    wi_v, ci_v, wq_v, cq_v, wr_v, cr_v, o_v, sem_w, sem_c):
    wid = lax.axis_index("s") * NUM_CORES + lax.axis_index("c")
    base = wid * BPW

    pltpu.sync_copy(word_hbm.at[pl.ds(base, BPW)], wi_v)
    pltpu.sync_copy(ctx_hbm.at[pl.ds(base, BPW)], ci_v)

    def shift_body(g, _):
        sl = pl.ds(g * LANES, LANES)
        wq_v[sl] = jax.lax.shift_right_logical(wi_v[sl], 2)
        cq_v[sl] = jax.lax.shift_right_logical(ci_v[sl], 2)
        return 0

    lax.fori_loop(0, BPW // LANES, shift_body, 0)

    for c in range(NCHUNKS):
        cbase = c * CHUNK
        cp_w = pltpu.async_copy(
            ht_hbm.at[wq_v.at[pl.ds(cbase, CHUNK)]], wr_v, sem_w)
        cp_c = pltpu.async_copy(
            ct_hbm.at[cq_v.at[pl.ds(cbase, CHUNK)]], cr_v, sem_c)
        cp_w.wait()
        cp_c.wait()

        def group_body(g, _):
            row0 = g * LANES
            wsub_g = wi_v[pl.ds(cbase + row0, LANES)] & 3
            csub_g = ci_v[pl.ds(cbase + row0, LANES)] & 3
            acc = jnp.zeros((LANES,), jnp.float32)
            for i in range(LANES):
                r = row0 + i
                wsub = wsub_g[i]
                csub = csub_g[i]
                wa, wb = _select_subrow(wr_v, r, wsub)
                ca, cb = _select_subrow(cr_v, r, csub)
                p = wa * ca + wb * cb
                for sh in (8, 4, 2, 1):
                    p = p + _permute(
                        p, jnp.bitwise_xor(lax.iota(jnp.int32, LANES), sh))
                acc = jnp.where(lax.iota(jnp.int32, LANES) == i, p, acc)
            o_v[pl.ds(cbase + row0, LANES)] = 1.0 / (1.0 + jnp.exp(-acc))
            return 0

        lax.fori_loop(0, CGROUPS, group_body, 0)

    pltpu.sync_copy(o_v, out_hbm.at[pl.ds(base, BPW)])


_sc_call = functools.partial(
    pl.kernel,
    out_type=jax.ShapeDtypeStruct((BATCH,), jnp.float32),
    mesh=plsc.VectorSubcoreMesh(
        core_axis_name="c", subcore_axis_name="s",
        num_cores=NUM_CORES, num_subcores=NUM_SUBCORES),
    compiler_params=pltpu.CompilerParams(use_tc_tiling_on_sc=True),
    scratch_types=[
        pltpu.VMEM((BPW,), jnp.int32),
        pltpu.VMEM((BPW,), jnp.int32),
        pltpu.VMEM((BPW,), jnp.int32),
        pltpu.VMEM((BPW,), jnp.int32),
        pltpu.VMEM((CHUNK, WIDE), jnp.float32),
        pltpu.VMEM((CHUNK, WIDE), jnp.float32),
        pltpu.VMEM((BPW,), jnp.float32),
        pltpu.SemaphoreType.DMA,
        pltpu.SemaphoreType.DMA,
    ],
)(_sc_kernel_body)


def kernel(word, context, hidden_table, context_table):
    out = _sc_call(word.astype(jnp.int32), context.astype(jnp.int32),
                   hidden_table.reshape(NWIDE, WIDE),
                   context_table.reshape(NWIDE, WIDE))
    return out.reshape(BATCH, 1)


# speedup vs baseline: 2.2086x; 1.0210x over previous
import functools

import jax
import jax.numpy as jnp
from jax import lax
from jax.experimental import pallas as pl
from jax.experimental.pallas import tpu as pltpu
from jax.experimental.pallas import tpu_sc as plsc

VOCAB = 1000000
DIM = 32
BATCH = 16384

NUM_CORES = 2
NUM_SUBCORES = 16
LANES = 16
NW = NUM_CORES * NUM_SUBCORES
BPW = BATCH // NW
GROUPS = BPW // LANES

_DNUMS = jax.lax.GatherDimensionNumbers(
    offset_dims=(), collapsed_slice_dims=(0,), start_index_map=(0,))


def _permute(v, idx):
    return jax.lax.gather(
        v, idx.reshape(LANES, 1), _DNUMS, slice_sizes=(1,),
        mode=jax.lax.GatherScatterMode.PROMISE_IN_BOUNDS)


def _sc_kernel_body(word_hbm, ctx_hbm, ht_hbm, ct_hbm, out_hbm,
                    wi_v, ci_v, wr_v, cr_v, o_v, sem_w, sem_c):
    wid = lax.axis_index("s") * NUM_CORES + lax.axis_index("c")
    base = wid * BPW

    pltpu.sync_copy(word_hbm.at[pl.ds(base, BPW)], wi_v)
    pltpu.sync_copy(ctx_hbm.at[pl.ds(base, BPW)], ci_v)
    cp_w = pltpu.async_copy(ht_hbm.at[wi_v], wr_v, sem_w)
    cp_c = pltpu.async_copy(ct_hbm.at[ci_v], cr_v, sem_c)
    cp_w.wait()
    cp_c.wait()

    def group_body(g, _):
        row0 = g * LANES
        acc = jnp.zeros((LANES,), jnp.float32)
        for i in range(LANES):
            r = row0 + i
            wa = wr_v[r, pl.ds(0, LANES)]
            wb = wr_v[r, pl.ds(LANES, LANES)]
            ca = cr_v[r, pl.ds(0, LANES)]
            cb = cr_v[r, pl.ds(LANES, LANES)]
            p = wa * ca + wb * cb
            for sh in (8, 4, 2, 1):
                p = p + _permute(p, jnp.bitwise_xor(lax.iota(jnp.int32, LANES), sh))
            acc = jnp.where(lax.iota(jnp.int32, LANES) == i, p, acc)
        o_v[pl.ds(row0, LANES)] = 1.0 / (1.0 + jnp.exp(-acc))
        return 0

    lax.fori_loop(0, GROUPS, group_body, 0)
    pltpu.sync_copy(o_v, out_hbm.at[pl.ds(base, BPW)])


_sc_call = functools.partial(
    pl.kernel,
    out_type=jax.ShapeDtypeStruct((BATCH,), jnp.float32),
    mesh=plsc.VectorSubcoreMesh(
        core_axis_name="c", subcore_axis_name="s",
        num_cores=NUM_CORES, num_subcores=NUM_SUBCORES),
    compiler_params=pltpu.CompilerParams(use_tc_tiling_on_sc=False),
    scratch_types=[
        pltpu.VMEM((BPW,), jnp.int32),
        pltpu.VMEM((BPW,), jnp.int32),
        pltpu.VMEM((BPW, DIM), jnp.float32),
        pltpu.VMEM((BPW, DIM), jnp.float32),
        pltpu.VMEM((BPW,), jnp.float32),
        pltpu.SemaphoreType.DMA,
        pltpu.SemaphoreType.DMA,
    ],
)(_sc_kernel_body)


def kernel(word, context, hidden_table, context_table):
    out = _sc_call(word.astype(jnp.int32), context.astype(jnp.int32),
                   hidden_table, context_table)
    return out.reshape(BATCH, 1)
